# Initial kernel scaffold; baseline (speedup 1.0000x reference)
import jax
import jax.numpy as jnp
from jax.experimental import pallas as pl


def kernel(x, edge_index, edge_weight, W, b):
    raise NotImplementedError("write your pallas kernel here")


# speedup vs baseline: 36.8276x; 36.8276x over previous
import functools

import jax
import jax.numpy as jnp
from jax import lax
from jax.experimental import pallas as pl
from jax.experimental.pallas import tpu as pltpu
from jax.experimental.pallas import tpu_sc as plsc

N = 10000
E = 320000
IN = 128
OUT = 16

NC = 2
NS = 16
NW = NC * NS
EPT = E // NW
CH = 80
NCHUNK = EPT // CH
GP = CH // 16
NPAD = 10240
RPT = NPAD // NS

_mesh = plsc.VectorSubcoreMesh(
    core_axis_name="c", subcore_axis_name="s", num_cores=NC, num_subcores=NS
)
_sc_params = pltpu.CompilerParams(needs_layout_passes=False,
                                  use_tc_tiling_on_sc=False)


@functools.partial(
    pl.kernel,
    out_type=jax.ShapeDtypeStruct((NW, NPAD // 128, 128), jnp.float32),
    mesh=_mesh,
    scratch_types=[
        pltpu.VMEM((NCHUNK, CH), jnp.int32),
        pltpu.VMEM((NCHUNK, CH), jnp.float32),
        pltpu.VMEM((NPAD // 128, 128), jnp.float32),
    ],
    compiler_params=_sc_params,
)
def _deg_kernel(dst_hbm, ew_hbm, out_hbm, dst_v, ew_v, deg_v):
    c = lax.axis_index("c")
    s = lax.axis_index("s")
    wid = s * NC + c
    pltpu.sync_copy(dst_hbm.at[wid], dst_v)
    pltpu.sync_copy(ew_hbm.at[wid], ew_v)

    def zero_body(i, _):
        deg_v[i // 8, pl.ds((i % 8) * 16, 16)] = jnp.zeros((16,), jnp.float32)
        return 0

    lax.fori_loop(0, NPAD // 16, zero_body, 0)

    def chunk_body(j, _):
        def grp_body(g, _):
            idx16 = dst_v[j, pl.ds(g * 16, 16)]
            w16 = ew_v[j, pl.ds(g * 16, 16)]
            plsc.addupdate_scatter(
                deg_v,
                [lax.shift_right_logical(idx16, 7),
                 lax.bitwise_and(idx16, 127)],
                w16,
            )
            return 0

        lax.fori_loop(0, GP, grp_body, 0)
        return 0

    lax.fori_loop(0, NCHUNK, chunk_body, 0)
    pltpu.sync_copy(deg_v, out_hbm.at[wid])


_RB = 1280
_GRID = NPAD // _RB


def _linear_body(x_ref, w_ref, pdeg_ref, b_ref, h2_ref, dis_ref, init_ref):
    deg = jnp.sum(pdeg_ref[...], axis=0) + 1.0
    safe = jnp.where(deg > 0, deg, 1.0)
    dis = jnp.where(deg > 0, lax.rsqrt(safe), 0.0)
    h = jnp.dot(x_ref[...], w_ref[...], preferred_element_type=jnp.float32,
                precision=lax.Precision.HIGHEST)
    h2 = h * dis[:, None]
    h2_ref[...] = h2
    dis_ref[...] = dis[None, None, :]
    init_ref[...] = h2 * dis[:, None] + b_ref[0, :]


def _linear(x, W, pdeg, b2):
    return pl.pallas_call(
        _linear_body,
        grid=(_GRID,),
        in_specs=[
            pl.BlockSpec((_RB, IN), lambda i: (i, 0)),
            pl.BlockSpec((IN, OUT), lambda i: (0, 0)),
            pl.BlockSpec((NW, _RB), lambda i: (0, i)),
            pl.BlockSpec((1, OUT), lambda i: (0, 0)),
        ],
        out_specs=[
            pl.BlockSpec((_RB, OUT), lambda i: (i, 0)),
            pl.BlockSpec((1, 1, _RB), lambda i: (i, 0, 0)),
            pl.BlockSpec((_RB, OUT), lambda i: (i, 0)),
        ],
        out_shape=[
            jax.ShapeDtypeStruct((N, OUT), jnp.float32),
            jax.ShapeDtypeStruct((_GRID, 1, _RB), jnp.float32),
            jax.ShapeDtypeStruct((N, OUT), jnp.float32),
        ],
    )(x, W, pdeg, b2)


@functools.partial(
    pl.kernel,
    out_type=jax.ShapeDtypeStruct((NC, NS, RPT, OUT), jnp.float32),
    mesh=_mesh,
    scratch_types=[
        pltpu.VMEM((NCHUNK, CH), jnp.int32),
        pltpu.VMEM((NCHUNK, CH), jnp.int32),
        pltpu.VMEM((NCHUNK, CH), jnp.float32),
        pltpu.VMEM((NPAD,), jnp.float32),
        pltpu.VMEM((CH, OUT), jnp.float32),
        pltpu.VMEM((RPT, OUT), jnp.float32),
        pltpu.VMEM_SHARED((NPAD, OUT), jnp.float32),
        pltpu.SemaphoreType.DMA,
    ],
    compiler_params=_sc_params,
)
def _edge_kernel(src_hbm, dst_hbm, ew_hbm, h2_hbm, dis_hbm, out_hbm,
                 src_v, dst_v, ew_v, dis_v, rows_v, z_v, acc_sh, sem):
    c = lax.axis_index("c")
    s = lax.axis_index("s")
    wid = s * NC + c

    def zero_body(i, _):
        z_v[i] = jnp.zeros((OUT,), jnp.float32)
        return 0

    lax.fori_loop(0, RPT, zero_body, 0)
    pltpu.sync_copy(z_v, acc_sh.at[pl.ds(s * RPT, RPT)])

    pltpu.sync_copy(src_hbm.at[wid], src_v)
    pltpu.sync_copy(dst_hbm.at[wid], dst_v)
    pltpu.sync_copy(ew_hbm.at[wid], ew_v)
    pltpu.sync_copy(dis_hbm, dis_v)
    plsc.subcore_barrier()

    def chunk_body(j, _):
        pltpu.async_copy(h2_hbm.at[src_v.at[j]], rows_v, sem).wait()

        def grp_body(g, _):
            base = g * 16
            d16 = dst_v[j, pl.ds(base, 16)]
            w16 = ew_v[j, pl.ds(base, 16)]
            s16 = plsc.load_gather(dis_v, [d16]) * w16
            for l in range(16):
                e = base + l
                rows_v[e] = rows_v[e] * s16[l]
            return 0

        lax.fori_loop(0, GP, grp_body, 0)
        pltpu.sync_copy(rows_v, acc_sh.at[dst_v.at[j]], add=True)
        return 0

    lax.fori_loop(0, NCHUNK, chunk_body, 0)
    plsc.subcore_barrier()
    pltpu.sync_copy(acc_sh.at[pl.ds(s * RPT, RPT)], out_hbm.at[c, s])


def _final_body(init_ref, parts_ref, o_ref):
    p = parts_ref[...].reshape(NC, _RB, OUT)
    o_ref[...] = init_ref[...] + p[0] + p[1]


def _final(out_init, parts):
    return pl.pallas_call(
        _final_body,
        grid=(_GRID,),
        in_specs=[
            pl.BlockSpec((_RB, OUT), lambda i: (i, 0)),
            pl.BlockSpec((NC, _RB // RPT, RPT, OUT), lambda i: (0, i, 0, 0)),
        ],
        out_specs=pl.BlockSpec((_RB, OUT), lambda i: (i, 0)),
        out_shape=jax.ShapeDtypeStruct((N, OUT), jnp.float32),
    )(out_init, parts)


def kernel(x, edge_index, edge_weight, W, b):
    src = edge_index[0].reshape(NW, NCHUNK, CH)
    dst = edge_index[1].reshape(NW, NCHUNK, CH)
    ew = edge_weight.reshape(NW, NCHUNK, CH)

    pdeg = _deg_kernel(dst, ew)
    h2, dis3, out_init = _linear(x, W, pdeg.reshape(NW, NPAD), b.reshape(1, OUT))
    parts = _edge_kernel(src, dst, ew, h2, dis3.reshape(NPAD))
    return _final(out_init, parts)


# speedup vs baseline: 51.5657x; 1.4002x over previous
import functools

import jax
import jax.numpy as jnp
from jax import lax
from jax.experimental import pallas as pl
from jax.experimental.pallas import tpu as pltpu
from jax.experimental.pallas import tpu_sc as plsc

N = 10000
E = 320000
IN = 128
OUT = 16

NC = 2
NS = 16
NW = NC * NS
EPT = E // NW
CH = 80
NCHUNK = EPT // CH
GP = CH // 16
NPAD = 10240
RPT = NPAD // NS
RING = 5

_mesh = plsc.VectorSubcoreMesh(
    core_axis_name="c", subcore_axis_name="s", num_cores=NC, num_subcores=NS
)
_sc_params = pltpu.CompilerParams(needs_layout_passes=False,
                                  use_tc_tiling_on_sc=False)


@functools.partial(
    pl.kernel,
    out_type=jax.ShapeDtypeStruct((NW, NPAD // 128, 128), jnp.float32),
    mesh=_mesh,
    scratch_types=[
        pltpu.VMEM((NCHUNK, CH), jnp.int32),
        pltpu.VMEM((NCHUNK, CH), jnp.float32),
        pltpu.VMEM((NPAD // 128, 128), jnp.float32),
    ],
    compiler_params=_sc_params,
)
def _deg_kernel(dst_hbm, ew_hbm, out_hbm, dst_v, ew_v, deg_v):
    c = lax.axis_index("c")
    s = lax.axis_index("s")
    wid = s * NC + c
    pltpu.sync_copy(dst_hbm.at[wid], dst_v)
    pltpu.sync_copy(ew_hbm.at[wid], ew_v)

    def zero_body(i, _):
        deg_v[i // 8, pl.ds((i % 8) * 16, 16)] = jnp.zeros((16,), jnp.float32)
        return 0

    lax.fori_loop(0, NPAD // 16, zero_body, 0)

    def chunk_body(j, _):
        def grp_body(g, _):
            idx16 = dst_v[j, pl.ds(g * 16, 16)]
            w16 = ew_v[j, pl.ds(g * 16, 16)]
            plsc.addupdate_scatter(
                deg_v,
                [lax.shift_right_logical(idx16, 7),
                 lax.bitwise_and(idx16, 127)],
                w16,
            )
            return 0

        lax.fori_loop(0, GP, grp_body, 0)
        return 0

    lax.fori_loop(0, NCHUNK, chunk_body, 0)
    pltpu.sync_copy(deg_v, out_hbm.at[wid])


_RB = 1280
_GRID = NPAD // _RB


def _linear_body(x_ref, w_ref, pdeg_ref, b_ref, h2_ref, dis_ref, init_ref):
    deg = jnp.sum(pdeg_ref[...], axis=0) + 1.0
    safe = jnp.where(deg > 0, deg, 1.0)
    dis = jnp.where(deg > 0, lax.rsqrt(safe), 0.0)
    h = jnp.dot(x_ref[...], w_ref[...], preferred_element_type=jnp.float32,
                precision=lax.Precision.HIGHEST)
    h2 = h * dis[:, None]
    h2_ref[...] = h2
    dis_ref[...] = dis[None, None, :]
    init_ref[...] = h2 * dis[:, None] + b_ref[0, :]


def _linear(x, W, pdeg, b2):
    return pl.pallas_call(
        _linear_body,
        grid=(_GRID,),
        in_specs=[
            pl.BlockSpec((_RB, IN), lambda i: (i, 0)),
            pl.BlockSpec((IN, OUT), lambda i: (0, 0)),
            pl.BlockSpec((NW, _RB), lambda i: (0, i)),
            pl.BlockSpec((1, OUT), lambda i: (0, 0)),
        ],
        out_specs=[
            pl.BlockSpec((_RB, OUT), lambda i: (i, 0)),
            pl.BlockSpec((1, 1, _RB), lambda i: (i, 0, 0)),
            pl.BlockSpec((_RB, OUT), lambda i: (i, 0)),
        ],
        out_shape=[
            jax.ShapeDtypeStruct((N, OUT), jnp.float32),
            jax.ShapeDtypeStruct((_GRID, 1, _RB), jnp.float32),
            jax.ShapeDtypeStruct((N, OUT), jnp.float32),
        ],
    )(x, W, pdeg, b2)


@functools.partial(
    pl.kernel,
    out_type=jax.ShapeDtypeStruct((NC, NS, RPT, OUT), jnp.float32),
    mesh=_mesh,
    scratch_types=[
        pltpu.VMEM((NCHUNK, CH), jnp.int32),
        pltpu.VMEM((NCHUNK, CH), jnp.int32),
        pltpu.VMEM((NCHUNK, CH), jnp.float32),
        pltpu.VMEM((NPAD,), jnp.float32),
        pltpu.VMEM((RING, CH, OUT), jnp.float32),
        pltpu.VMEM((RING, CH, OUT), jnp.float32),
        pltpu.VMEM((RPT, OUT), jnp.float32),
        pltpu.VMEM_SHARED((NPAD, OUT), jnp.float32),
        pltpu.SemaphoreType.DMA((RING,)),
        pltpu.SemaphoreType.DMA((RING,)),
    ],
    compiler_params=_sc_params,
)
def _edge_kernel(src_hbm, dst_hbm, ew_hbm, h2_hbm, dis_hbm, out_hbm,
                 src_v, dst_v, ew_v, dis_v, grow_v, srow_v, z_v, acc_sh,
                 gsem, ssem):
    c = lax.axis_index("c")
    s = lax.axis_index("s")
    wid = s * NC + c

    def zero_body(i, _):
        z_v[i] = jnp.zeros((OUT,), jnp.float32)
        return 0

    lax.fori_loop(0, RPT, zero_body, 0)
    pltpu.sync_copy(z_v, acc_sh.at[pl.ds(s * RPT, RPT)])

    pltpu.sync_copy(src_hbm.at[wid], src_v)
    pltpu.sync_copy(dst_hbm.at[wid], dst_v)
    pltpu.sync_copy(ew_hbm.at[wid], ew_v)
    pltpu.sync_copy(dis_hbm, dis_v)
    plsc.subcore_barrier()

    for b in range(RING - 1):
        pltpu.async_copy(h2_hbm.at[src_v.at[b]], grow_v.at[b], gsem.at[b])

    def outer_body(o, _):
        for b in range(RING):
            j = o * RING + b
            pltpu.make_async_copy(
                h2_hbm.at[src_v.at[j]], grow_v.at[b], gsem.at[b]).wait()

            @pl.when(o > 0)
            def _():
                pltpu.make_async_copy(
                    srow_v.at[b], acc_sh.at[dst_v.at[j]], ssem.at[b]).wait()

            def grp_body(g, _):
                base = g * 16
                d16 = dst_v[j, pl.ds(base, 16)]
                w16 = ew_v[j, pl.ds(base, 16)]
                s16 = plsc.load_gather(dis_v, [d16]) * w16
                for l in range(16):
                    e = base + l
                    srow_v[b, e] = grow_v[b, e] * s16[l]
                return 0

            lax.fori_loop(0, GP, grp_body, 0)
            pltpu.async_copy(srow_v.at[b], acc_sh.at[dst_v.at[j]],
                             ssem.at[b], add=True)

            nxt = j + RING - 1
            nb = (b + RING - 1) % RING

            @pl.when(nxt < NCHUNK)
            def _():
                pltpu.async_copy(h2_hbm.at[src_v.at[nxt]], grow_v.at[nb],
                                 gsem.at[nb])
        return 0

    lax.fori_loop(0, NCHUNK // RING, outer_body, 0)
    for b in range(RING):
        pltpu.make_async_copy(
            srow_v.at[b], acc_sh.at[dst_v.at[0]], ssem.at[b]).wait()
    plsc.subcore_barrier()
    pltpu.sync_copy(acc_sh.at[pl.ds(s * RPT, RPT)], out_hbm.at[c, s])


def _final_body(init_ref, parts_ref, o_ref):
    p = parts_ref[...].reshape(NC, _RB, OUT)
    o_ref[...] = init_ref[...] + p[0] + p[1]


def _final(out_init, parts):
    return pl.pallas_call(
        _final_body,
        grid=(_GRID,),
        in_specs=[
            pl.BlockSpec((_RB, OUT), lambda i: (i, 0)),
            pl.BlockSpec((NC, _RB // RPT, RPT, OUT), lambda i: (0, i, 0, 0)),
        ],
        out_specs=pl.BlockSpec((_RB, OUT), lambda i: (i, 0)),
        out_shape=jax.ShapeDtypeStruct((N, OUT), jnp.float32),
    )(out_init, parts)


def kernel(x, edge_index, edge_weight, W, b):
    src = edge_index[0].reshape(NW, NCHUNK, CH)
    dst = edge_index[1].reshape(NW, NCHUNK, CH)
    ew = edge_weight.reshape(NW, NCHUNK, CH)

    pdeg = _deg_kernel(dst, ew)
    h2, dis3, out_init = _linear(x, W, pdeg.reshape(NW, NPAD), b.reshape(1, OUT))
    parts = _edge_kernel(src, dst, ew, h2, dis3.reshape(NPAD))
    return _final(out_init, parts)
